# baseline (speedup 1.0000x reference)
import math

import jax
import jax.numpy as jnp
from jax import lax
from jax.experimental import pallas as pl
from jax.experimental.pallas import tpu as pltpu
from jax.experimental.pallas import tpu_sc as plsc

D_MODEL = 512
SCALE = math.sqrt(D_MODEL)

NUM_CORES = 2
NUM_SUBCORES = 16
NUM_LANES = 16
NW = NUM_CORES * NUM_SUBCORES

GROUP = 2


def _sc_embedding(x, table):
    b, t = x.shape
    assert b % (NW * GROUP) == 0 and (GROUP * t) % 8 == 0
    b_per_w = b // NW
    n_groups = b_per_w // GROUP
    gsz = GROUP * t
    mesh = plsc.VectorSubcoreMesh(core_axis_name="c", subcore_axis_name="s")

    def body(idx_hbm, table_hbm, out_hbm,
             idx_v, raw0, raw1, stg0, stg1, sem0, sem1):
        wid = lax.axis_index("s") * NUM_CORES + lax.axis_index("c")
        pltpu.sync_copy(idx_hbm.at[wid], idx_v)
        base = wid * b_per_w
        last = n_groups - 1

        def start(g, raw, sem):
            pltpu.make_async_copy(
                table_hbm.at[idx_v.at[pl.ds(g * gsz, gsz)]], raw, sem
            ).start()

        def drain(raw, sem):
            pltpu.make_async_copy(
                table_hbm.at[idx_v.at[pl.ds(0, gsz)]], raw, sem
            ).wait()

        def scale_store(g, raw, stg):
            def scale_row(r, _):
                for j in range(GROUP):
                    for c in range(D_MODEL // NUM_LANES):
                        sl = pl.ds(c * NUM_LANES, NUM_LANES)
                        stg[j, r, sl] = raw[j * t + r, sl] * SCALE
                return 0

            lax.fori_loop(0, t, scale_row, 0)
            pltpu.sync_copy(stg, out_hbm.at[pl.ds(base + g * GROUP, GROUP)])

        start(0, raw0, sem0)

        def ring(i, _):
            g0 = 2 * i
            g1 = g0 + 1
            start(g1, raw1, sem1)
            drain(raw0, sem0)
            scale_store(g0, raw0, stg0)
            start(jnp.minimum(g1 + 1, last), raw0, sem0)
            drain(raw1, sem1)
            scale_store(g1, raw1, stg1)
            return 0

        lax.fori_loop(0, n_groups // 2, ring, 0)
        drain(raw0, sem0)

    run = pl.kernel(
        body,
        out_type=jax.ShapeDtypeStruct((b, t, D_MODEL), jnp.float32),
        mesh=mesh,
        scratch_types=[
            pltpu.VMEM((b_per_w * t,), jnp.int32),
            pltpu.VMEM((gsz, D_MODEL), jnp.float32),
            pltpu.VMEM((gsz, D_MODEL), jnp.float32),
            pltpu.VMEM((GROUP, t, D_MODEL), jnp.float32),
            pltpu.VMEM((GROUP, t, D_MODEL), jnp.float32),
            pltpu.SemaphoreType.DMA,
            pltpu.SemaphoreType.DMA,
        ],
    )
    idx2 = x.astype(jnp.int32).reshape(NW, b_per_w * t)
    return run(idx2, table)


def kernel(x, table):
    return _sc_embedding(x, table)
